# Initial kernel scaffold; baseline (speedup 1.0000x reference)
import functools

import jax
import jax.numpy as jnp
from jax import lax
from jax.experimental import pallas as pl
from jax.experimental.pallas import tpu as pltpu
from jax.experimental.pallas import tpu_sc as plsc

_N_NODES = 10000
_N_EDGES = 50000
_NNZ = 320000
_D = 128

_NC = 2
_NS = 16
_L = 16
_B = 128

_NNZ_T = _NNZ // _NS
_PIECE = 2000
_NPIECE = _NNZ_T // _PIECE
_VPP = _PIECE // _L
_MAXB = (_NNZ_T + _B - 1) // _B + 1


def _build_scatter_mean(n_src, n_out, cpc):
  ch = n_out // (_NC * cpc)
  assert ch * _NC * cpc == n_out
  rpt = -(-ch // _NS)
  zr = -(-(ch + 1) // _NS)
  assert rpt >= 64 and zr >= 64

  mesh = plsc.VectorSubcoreMesh(core_axis_name="c", subcore_axis_name="s")

  def body(src, gat, seg, out,
           gi_v, si_v, bgi, bsi, rows, onesb, zrow, zcnt, sums64, out64,
           cnt64, ssum, scnt, sem):
    cid = lax.axis_index("c")
    sid = lax.axis_index("s")

    @pl.loop(0, 64)
    def _init_z(r):
      zcnt[r] = jnp.zeros((_L,), jnp.float32)
      for g in range(_D // _L):
        zrow[r, pl.ds(g * _L, _L)] = jnp.zeros((_L,), jnp.float32)

    @pl.loop(0, _B)
    def _init_o(r):
      onesb[r] = jnp.ones((_L,), jnp.float32)

    row_lo = sid * rpt
    row_hi = jnp.minimum(row_lo + rpt, ch)

    for rnd in range(cpc):
      cb = (cid * cpc + rnd) * ch

      z0 = sid * zr
      z1 = jnp.minimum(z0 + zr, ch + 1)
      nzfull = (z1 - z0) // 64

      @pl.loop(0, nzfull)
      def _zb(k):
        pltpu.sync_copy(zrow, ssum.at[pl.ds(z0 + k * 64, 64)])
        pltpu.sync_copy(zcnt, scnt.at[pl.ds(z0 + k * 64, 64)])

      @pl.when((z1 - z0) % 64 != 0)
      def _zt():
        pltpu.sync_copy(zrow, ssum.at[pl.ds(z1 - 64, 64)])
        pltpu.sync_copy(zcnt, scnt.at[pl.ds(z1 - 64, 64)])

      plsc.subcore_barrier()

      @pl.loop(0, _NPIECE, init_carry=jnp.int32(0))
      def cnt(p, c):
        off = sid * _NNZ_T + p * _PIECE
        pltpu.sync_copy(gat.at[pl.ds(off, _PIECE)], gi_v)
        pltpu.sync_copy(seg.at[pl.ds(off, _PIECE)], si_v)

        @pl.loop(0, _VPP, init_carry=c)
        def c2(t, cc):
          e = si_v[pl.ds(t * _L, _L)]
          v = gi_v[pl.ds(t * _L, _L)]
          m = (e >= cb) & (e < cb + ch)
          mi = m.astype(jnp.int32)
          pos = cc + plsc.cumsum(mi) - 1
          plsc.store_scatter(bgi, [pos >> 7, pos & 127], v, mask=m)
          plsc.store_scatter(bsi, [pos >> 7, pos & 127], e - cb, mask=m)
          return cc + jnp.sum(mi)

        return c2

      pad_end = ((cnt + _B - 1) >> 7) << 7
      io = lax.iota(jnp.int32, _L)
      for t in range(_B // _L):
        pos = cnt + t * _L + io
        m = pos < pad_end
        plsc.store_scatter(bgi, [pos >> 7, pos & 127],
                           jnp.zeros((_L,), jnp.int32), mask=m)
        plsc.store_scatter(bsi, [pos >> 7, pos & 127],
                           jnp.full((_L,), ch, jnp.int32), mask=m)
      nblk = pad_end >> 7

      @pl.loop(0, nblk)
      def _g(j):
        pltpu.async_copy(src.at[bgi.at[j]], rows, sem).wait()
        pltpu.sync_copy(rows, ssum.at[bsi.at[j]], add=True)
        pltpu.sync_copy(onesb, scnt.at[bsi.at[j]], add=True)

      plsc.subcore_barrier()

      def fin_block(r0):
        pltpu.sync_copy(ssum.at[pl.ds(r0, 64)], sums64)
        pltpu.sync_copy(scnt.at[pl.ds(r0, 64)], cnt64)

        @pl.loop(0, 64)
        def _c(r):
          rec = 1.0 / jnp.maximum(cnt64[r], 1.0)
          for g in range(_D // _L):
            out64[r, pl.ds(g * _L, _L)] = sums64[r, pl.ds(g * _L, _L)] * rec

        pltpu.sync_copy(out64, out.at[pl.ds(cb + r0, 64)])

      nffull = (row_hi - row_lo) // 64

      @pl.loop(0, nffull)
      def _f(k):
        fin_block(row_lo + k * 64)

      @pl.when((row_hi - row_lo) % 64 != 0)
      def _ft():
        fin_block(row_hi - 64)

      plsc.subcore_barrier()

  return pl.kernel(
      body,
      out_type=jax.ShapeDtypeStruct((n_out, _D), jnp.float32),
      mesh=mesh,
      scratch_types=[
          pltpu.VMEM((_PIECE,), jnp.int32),
          pltpu.VMEM((_PIECE,), jnp.int32),
          pltpu.VMEM((_MAXB, _B), jnp.int32),
          pltpu.VMEM((_MAXB, _B), jnp.int32),
          pltpu.VMEM((_B, _D), jnp.float32),
          pltpu.VMEM((_B, _L), jnp.float32),
          pltpu.VMEM((64, _D), jnp.float32),
          pltpu.VMEM((64, _L), jnp.float32),
          pltpu.VMEM((64, _D), jnp.float32),
          pltpu.VMEM((64, _D), jnp.float32),
          pltpu.VMEM((64, _L), jnp.float32),
          pltpu.VMEM_SHARED((ch + 1, _D), jnp.float32),
          pltpu.VMEM_SHARED((ch + 1, _L), jnp.float32),
          pltpu.SemaphoreType.DMA,
      ],
  )


_PASS_E = _build_scatter_mean(_N_NODES, _N_EDGES, cpc=2)
_PASS_V = _build_scatter_mean(_N_EDGES, _N_NODES, cpc=1)

_M_BLK = 1000


def _linear_body(ab_ref, xv_ref, x0_ref, w_ref, o_ref):
  a = ab_ref[0, 0]
  b = ab_ref[0, 1]
  xi = (1.0 - a) * xv_ref[...] + a * x0_ref[...]
  xw = lax.dot_general(xi, w_ref[...], (((1,), (1,)), ((), ())),
                       preferred_element_type=jnp.float32)
  o_ref[...] = (1.0 - b) * xi + b * xw


_LINEAR = pl.pallas_call(
    _linear_body,
    grid=(_N_NODES // _M_BLK,),
    in_specs=[
        pl.BlockSpec((1, 2), lambda i: (0, 0)),
        pl.BlockSpec((_M_BLK, _D), lambda i: (i, 0)),
        pl.BlockSpec((_M_BLK, _D), lambda i: (i, 0)),
        pl.BlockSpec((_D, _D), lambda i: (0, 0)),
    ],
    out_specs=pl.BlockSpec((_M_BLK, _D), lambda i: (i, 0)),
    out_shape=jax.ShapeDtypeStruct((_N_NODES, _D), jnp.float32),
)


def kernel(X, vertex, edges, alpha, beta, X0, W):
  vertex = vertex.astype(jnp.int32)
  edges = edges.astype(jnp.int32)
  xe = _PASS_E(X, vertex, edges)
  xv = _PASS_V(xe, edges, vertex)
  ab = jnp.stack([alpha.astype(jnp.float32),
                  beta.astype(jnp.float32)]).reshape(1, 2)
  return _LINEAR(ab, xv, X0, W)


# speedup vs baseline: 2.1459x; 2.1459x over previous
import functools

import jax
import jax.numpy as jnp
from jax import lax
from jax.experimental import pallas as pl
from jax.experimental.pallas import tpu as pltpu
from jax.experimental.pallas import tpu_sc as plsc

_N_NODES = 10000
_N_EDGES = 50000
_NNZ = 320000
_D = 128

_NC = 2
_NS = 16
_L = 16
_B = 128

_NNZ_T = _NNZ // _NS
_PIECE = 2000
_NPIECE = _NNZ_T // _PIECE
_VPP = _PIECE // _L

_RB = 18
_TRASH = (_RB - 1) * _B


@functools.cache
def _build_scatter_sum(n_src, ch, cpc):
  assert ch % 128 == 0
  n_out = ch * _NC * cpc
  rpt = ch // _NS
  assert rpt % 8 == 0 and rpt >= 64

  mesh = plsc.VectorSubcoreMesh(core_axis_name="c", subcore_axis_name="s",
                                num_cores=_NC, num_subcores=_NS)

  def body(src, gat, seg, osum, ocnt,
           gi_v, si_v, bgi, bsi, rows, ones128, ssum, scnt, sem):
    cid = lax.axis_index("c")
    sid = lax.axis_index("s")

    @pl.loop(0, _B)
    def _init_o(r):
      for g in range(_D // _L):
        ones128[r, pl.ds(g * _L, _L)] = jnp.ones((_L,), jnp.float32)

    row_lo = sid * rpt
    nzb = rpt // 64
    has_tail = rpt % 64 != 0

    def flush(nblk):
      @pl.loop(0, nblk)
      def _g(j):
        pltpu.async_copy(src.at[bgi.at[j]], rows, sem).wait()
        pltpu.sync_copy(rows, ssum.at[bsi.at[j]], add=True)
        pltpu.sync_copy(ones128, scnt.at[bsi.at[j]], add=True)

    for rnd in range(cpc):
      cb = (cid * cpc + rnd) * ch

      @pl.loop(0, 64)
      def _z0(r):
        for g in range(_D // _L):
          rows[r, pl.ds(g * _L, _L)] = jnp.zeros((_L,), jnp.float32)

      @pl.loop(0, nzb)
      def _zb(k):
        pltpu.sync_copy(rows.at[pl.ds(0, 64)],
                        ssum.at[pl.ds(row_lo + k * 64, 64)])
        pltpu.sync_copy(rows.at[pl.ds(0, 64)],
                        scnt.at[pl.ds(row_lo + k * 64, 64)])

      if has_tail:
        pltpu.sync_copy(rows.at[pl.ds(0, 64)],
                        ssum.at[pl.ds(row_lo + rpt - 64, 64)])
        pltpu.sync_copy(rows.at[pl.ds(0, 64)],
                        scnt.at[pl.ds(row_lo + rpt - 64, 64)])

      plsc.subcore_barrier()

      @pl.loop(0, _NPIECE, init_carry=jnp.int32(0))
      def carry(p, cc0):
        off = sid * _NNZ_T + p * _PIECE
        pltpu.sync_copy(gat.at[pl.ds(off, _PIECE)], gi_v)
        pltpu.sync_copy(seg.at[pl.ds(off, _PIECE)], si_v)

        @pl.loop(0, _VPP, init_carry=cc0)
        def cc(t, c):
          e = si_v[pl.ds(t * _L, _L)]
          v = gi_v[pl.ds(t * _L, _L)]
          u = e - jnp.full((_L,), cb, jnp.int32)
          mi = ((u | (ch - 1 - u)) >> 31) + 1
          pos = c + plsc.cumsum(mi) - 1
          pos = pos * mi + _TRASH * (1 - mi)
          plsc.store_scatter(bgi, [pos >> 7, pos & 127], v)
          plsc.store_scatter(bsi, [pos >> 7, pos & 127], u)
          return c + jnp.sum(mi)

        nfb = cc >> 7
        flush(nfb)
        for g in range(_B // _L):
          bgi[0, pl.ds(g * _L, _L)] = bgi[nfb, pl.ds(g * _L, _L)]
          bsi[0, pl.ds(g * _L, _L)] = bsi[nfb, pl.ds(g * _L, _L)]
        return cc & 127

      io = lax.iota(jnp.int32, _L)
      for t in range(_B // _L):
        pos = carry + t * _L + io
        mi = -((pos - _B) >> 31)
        pos = pos * mi + _TRASH * (1 - mi)
        plsc.store_scatter(bgi, [pos >> 7, pos & 127],
                           jnp.zeros((_L,), jnp.int32))
        plsc.store_scatter(bsi, [pos >> 7, pos & 127],
                           jnp.full((_L,), ch, jnp.int32))
      flush(jnp.int32(1))

      plsc.subcore_barrier()

      def fin_block(r0):
        pltpu.sync_copy(ssum.at[pl.ds(r0, 64)], rows.at[pl.ds(0, 64)])
        pltpu.sync_copy(rows.at[pl.ds(0, 64)], osum.at[pl.ds(cb + r0, 64)])
        pltpu.sync_copy(scnt.at[pl.ds(r0, 64)], rows.at[pl.ds(64, 64)])
        pltpu.sync_copy(rows.at[pl.ds(64, 64)], ocnt.at[pl.ds(cb + r0, 64)])

      @pl.loop(0, nzb)
      def _f(k):
        fin_block(row_lo + k * 64)

      if has_tail:
        fin_block(row_lo + rpt - 64)

      plsc.subcore_barrier()

  return pl.kernel(
      body,
      out_type=[jax.ShapeDtypeStruct((n_out, _D), jnp.float32),
                jax.ShapeDtypeStruct((n_out, _D), jnp.float32)],
      mesh=mesh,
      compiler_params=pltpu.CompilerParams(needs_layout_passes=False),
      scratch_types=[
          pltpu.VMEM((_PIECE,), jnp.int32),
          pltpu.VMEM((_PIECE,), jnp.int32),
          pltpu.VMEM((_RB, _B), jnp.int32),
          pltpu.VMEM((_RB, _B), jnp.int32),
          pltpu.VMEM((_B, _D), jnp.float32),
          pltpu.VMEM((_B, _D), jnp.float32),
          pltpu.VMEM_SHARED((ch + 64, _D), jnp.float32),
          pltpu.VMEM_SHARED((ch + 64, _D), jnp.float32),
          pltpu.SemaphoreType.DMA,
      ],
  )


def _div_body(s_ref, c_ref, o_ref):
  c = jnp.maximum(c_ref[...][:, 0:1], 1.0)
  o_ref[...] = s_ref[...] / c


@functools.cache
def _build_div(n, blk):
  return pl.pallas_call(
      _div_body,
      grid=(n // blk,),
      in_specs=[
          pl.BlockSpec((blk, _D), lambda i: (i, 0)),
          pl.BlockSpec((blk, _D), lambda i: (i, 0)),
      ],
      out_specs=pl.BlockSpec((blk, _D), lambda i: (i, 0)),
      out_shape=jax.ShapeDtypeStruct((n, _D), jnp.float32),
  )


_M_BLK = 1000


def _linear_body(ab_ref, sv_ref, cv_ref, x0_ref, w_ref, o_ref):
  a = ab_ref[0, 0]
  b = ab_ref[0, 1]
  xv = sv_ref[...] / jnp.maximum(cv_ref[...][:, 0:1], 1.0)
  xi = (1.0 - a) * xv + a * x0_ref[...]
  xw = lax.dot_general(xi, w_ref[...], (((1,), (1,)), ((), ())),
                       preferred_element_type=jnp.float32)
  o_ref[...] = (1.0 - b) * xi + b * xw


_LINEAR = pl.pallas_call(
    _linear_body,
    grid=(_N_NODES // _M_BLK,),
    in_specs=[
        pl.BlockSpec(memory_space=pltpu.SMEM),
        pl.BlockSpec((_M_BLK, _D), lambda i: (i, 0)),
        pl.BlockSpec((_M_BLK, _D), lambda i: (i, 0)),
        pl.BlockSpec((_M_BLK, _D), lambda i: (i, 0)),
        pl.BlockSpec((_D, _D), lambda i: (0, 0)),
    ],
    out_specs=pl.BlockSpec((_M_BLK, _D), lambda i: (i, 0)),
    out_shape=jax.ShapeDtypeStruct((_N_NODES, _D), jnp.float32),
)


def kernel(X, vertex, edges, alpha, beta, X0, W):
  vertex = vertex.astype(jnp.int32)
  edges = edges.astype(jnp.int32)
  se, ce = _build_scatter_sum(_N_NODES, 2560, 10)(X, vertex, edges)
  xe = _build_div(se.shape[0], 1024)(se, ce)
  sv, cv = _build_scatter_sum(xe.shape[0], 2560, 2)(xe, edges, vertex)
  ab = jnp.stack([alpha.astype(jnp.float32),
                  beta.astype(jnp.float32)]).reshape(1, 2)
  return _LINEAR(ab, sv, cv, X0, W)
